# baseline (speedup 1.0000x reference)
import functools

import numpy as np
import jax
import jax.numpy as jnp
from jax import lax
from jax.experimental import pallas as pl
from jax.experimental.pallas import tpu as pltpu
from jax.experimental.pallas import tpu_sc as plsc

_N_BINS = 10
_LANES = 128
_GROUP = 128
_SUPER = 8
_BLK = 16384

_BOUNDS = [float(b) for b in np.linspace(0.0, 1.0, _N_BINS + 1).astype(np.float32)]


def _conf_kernel(x_ref, lab_ref, conf_ref, acc_ref, *, n_total, n_cols):
    i = pl.program_id(0)
    sub8 = jax.lax.broadcasted_iota(jnp.int32, (8, _LANES), 0).astype(jnp.float32)
    ridx = (jax.lax.broadcasted_iota(jnp.int32, (_SUPER, _LANES), 0) * _GROUP
            + jax.lax.broadcasted_iota(jnp.int32, (_SUPER, _LANES), 1))
    offs = list(range(8, n_cols - 8, 8)) + [n_cols - 8]

    for sg in range(_BLK // (_GROUP * _SUPER)):
        confs = []
        accs = []
        for g in range(_SUPER):
            c0 = (sg * _SUPER + g) * _GROUP
            m = jnp.exp(x_ref[0:8, c0:c0 + _GROUP])
            s = m
            idx = jnp.zeros((8, _LANES), jnp.float32)
            for off in offs[:-1]:
                ek = jnp.exp(x_ref[off:off + 8, c0:c0 + _GROUP])
                cond = ek > m
                m = jnp.maximum(m, ek)
                idx = jnp.where(cond, jnp.float32(off), idx)
                s = s + ek
            off = offs[-1]
            ek = jnp.exp(x_ref[off:off + 8, c0:c0 + _GROUP])
            cond = ek > m
            m = jnp.maximum(m, ek)
            idx = jnp.where(cond, jnp.float32(off), idx)
            s = s + jnp.where(sub8 >= jnp.float32(offs[-2] + 8 - off), ek, 0.0)
            cls = idx + sub8
            for sh in (4, 2, 1):
                mr = pltpu.roll(m, sh, 0)
                cr = pltpu.roll(cls, sh, 0)
                cond = mr > m
                m = jnp.maximum(m, mr)
                cls = jnp.where(cond, cr, cls)
                sr = pltpu.roll(s, sh, 0)
                s = s + sr
            labf = lab_ref[pl.ds(c0, _GROUP)].reshape(1, _GROUP).astype(jnp.float32)
            confs.append(m[0:1, :] / s[0:1, :])
            accs.append(jnp.where(cls[0:1, :] == labf, 1.0, 0.0))
        conf8 = jnp.concatenate(confs, axis=0)
        acc8 = jnp.concatenate(accs, axis=0)
        base = i * _BLK + sg * _GROUP * _SUPER
        valid = (base + ridx) < n_total
        conf_ref[sg] = jnp.where(valid, conf8, 0.0)
        acc_ref[sg] = acc8


_CH_INNER = 16


_CHUNK = 1984


def _sc_hist(conf_hbm, acc_hbm, out_hbm, cbuf, abuf, pbuf, *, total):
    nw = 32
    ch = total // nw
    wid = lax.axis_index("s") * 2 + lax.axis_index("c")
    base = wid * ch
    accs = tuple(jnp.zeros((_CH_INNER,), jnp.float32) for _ in range(3 * _N_BINS))
    for j in range(ch // _CHUNK):
        pltpu.sync_copy(conf_hbm.at[pl.ds(base + j * _CHUNK, _CHUNK)], cbuf)
        pltpu.sync_copy(acc_hbm.at[pl.ds(base + j * _CHUNK, _CHUNK)], abuf)

        def body(v, accs):
            c = cbuf[pl.ds(v * _CH_INNER, _CH_INNER)]
            a = abuf[pl.ds(v * _CH_INNER, _CH_INNER)]
            out = list(accs)
            for k in range(_N_BINS):
                gt = c > _BOUNDS[k]
                out[3 * k] = out[3 * k] + jnp.where(gt, 1.0, 0.0)
                out[3 * k + 1] = out[3 * k + 1] + jnp.where(gt, c, 0.0)
                out[3 * k + 2] = out[3 * k + 2] + jnp.where(gt, a, 0.0)
            return tuple(out)

        accs = lax.fori_loop(0, _CHUNK // _CH_INNER, body, accs)
    for r in range(3 * _N_BINS):
        pbuf[r, :] = accs[r]
    pltpu.sync_copy(pbuf, out_hbm.at[wid])


def _epi_kernel(part_ref, ece_ref, accb_ref, oe_ref, prop_ref, ce_ref, *, n_total):
    x = part_ref[...]
    nf = jnp.float32(n_total)
    zero = jnp.zeros((1, 1), jnp.float32)
    gc, gs, ga = [], [], []
    for k in range(_N_BINS):
        gc.append(jnp.sum(x[:, 3 * k, :], keepdims=True)[:1, :1])
        gs.append(jnp.sum(x[:, 3 * k + 1, :], keepdims=True)[:1, :1])
        ga.append(jnp.sum(x[:, 3 * k + 2, :], keepdims=True)[:1, :1])
    gc.append(zero)
    gs.append(zero)
    ga.append(zero)
    ece = zero
    oe = zero
    for b in range(_N_BINS):
        cnt = gc[b] - gc[b + 1]
        sc = gs[b] - gs[b + 1]
        sa = ga[b] - ga[b + 1]
        prop = cnt / nf
        denom = jnp.maximum(cnt, 1.0)
        accb = sa / denom
        avgc = sc / denom
        ce = avgc - accb
        ece = ece + jnp.abs(ce) * prop
        oe = oe + avgc * jnp.maximum(ce, 0.0) * prop
        accb_ref[pl.ds(b, 1), :] = accb
        prop_ref[pl.ds(b, 1), :] = prop
        ce_ref[pl.ds(b, 1), :] = jnp.abs(ce)
    ece_ref[...] = ece
    oe_ref[...] = oe


def kernel(logits, labels):
    n, c = logits.shape
    steps = pl.cdiv(n, _BLK)
    lt = logits.T
    nsg = _BLK // (_GROUP * _SUPER)

    conf_t, acc_t = pl.pallas_call(
        functools.partial(_conf_kernel, n_total=n, n_cols=c),
        grid=(steps,),
        in_specs=[
            pl.BlockSpec((c, _BLK), lambda i: (0, i)),
            pl.BlockSpec((_BLK,), lambda i: (i,)),
        ],
        out_specs=[
            pl.BlockSpec((nsg, 8, _LANES), lambda i: (i, 0, 0)),
            pl.BlockSpec((nsg, 8, _LANES), lambda i: (i, 0, 0)),
        ],
        out_shape=[
            jax.ShapeDtypeStruct((steps * nsg, 8, _LANES), jnp.float32),
            jax.ShapeDtypeStruct((steps * nsg, 8, _LANES), jnp.float32),
        ],
        compiler_params=pltpu.CompilerParams(dimension_semantics=("arbitrary",)),
    )(lt, labels)

    total = steps * nsg * 8 * _LANES
    conf_f = conf_t.reshape(total)
    acc_f = acc_t.reshape(total)

    mesh = plsc.VectorSubcoreMesh(core_axis_name="c", subcore_axis_name="s")
    parts = functools.partial(
        pl.kernel, mesh=mesh,
        out_type=jax.ShapeDtypeStruct((32, 3 * _N_BINS, _CH_INNER), jnp.float32),
        scratch_types=[
            pltpu.VMEM((_CHUNK,), jnp.float32),
            pltpu.VMEM((_CHUNK,), jnp.float32),
            pltpu.VMEM((3 * _N_BINS, _CH_INNER), jnp.float32),
        ],
    )(functools.partial(_sc_hist, total=total))(conf_f, acc_f)

    outs = pl.pallas_call(
        functools.partial(_epi_kernel, n_total=n),
        out_specs=[
            pl.BlockSpec((1, 1), lambda: (0, 0)),
            pl.BlockSpec((_N_BINS, 1), lambda: (0, 0)),
            pl.BlockSpec((1, 1), lambda: (0, 0)),
            pl.BlockSpec((_N_BINS, 1), lambda: (0, 0)),
            pl.BlockSpec((_N_BINS, 1), lambda: (0, 0)),
        ],
        out_shape=[
            jax.ShapeDtypeStruct((1, 1), jnp.float32),
            jax.ShapeDtypeStruct((_N_BINS, 1), jnp.float32),
            jax.ShapeDtypeStruct((1, 1), jnp.float32),
            jax.ShapeDtypeStruct((_N_BINS, 1), jnp.float32),
            jax.ShapeDtypeStruct((_N_BINS, 1), jnp.float32),
        ],
    )(parts)
    ece, accb, oe, prop, ce = outs
    return (ece.reshape(()), accb[:, 0], oe.reshape(()), prop[:, 0], ce[:, 0])


# speedup vs baseline: 1.5017x; 1.5017x over previous
import functools

import jax
import jax.numpy as jnp
from jax.experimental import pallas as pl
from jax.experimental.pallas import tpu as pltpu

_N_BINS = 10
_LANES = 128
_GROUP = 128
_SUPER = 8
_BLK = 49152


def _ece_kernel(bounds_ref, x_ref, lab_ref,
                ece_ref, accb_ref, oe_ref, prop_ref, ce_ref,
                cnt_ref, sc_ref, sa_ref, confrow_ref, accrow_ref,
                *, n_total, n_cols, n_steps):
    i = pl.program_id(0)

    @pl.when(i == 0)
    def _init():
        z = jnp.zeros((_N_BINS, 8, _LANES), jnp.float32)
        cnt_ref[...] = z
        sc_ref[...] = z
        sa_ref[...] = z

    sub8 = jax.lax.broadcasted_iota(jnp.int32, (8, _LANES), 0).astype(jnp.float32)
    ridx = (jax.lax.broadcasted_iota(jnp.int32, (_SUPER, _LANES), 0) * _GROUP
            + jax.lax.broadcasted_iota(jnp.int32, (_SUPER, _LANES), 1))
    offs = list(range(8, n_cols - 8, 8)) + [n_cols - 8]

    for sg in range(_BLK // (_GROUP * _SUPER)):
        for g in range(_SUPER):
            c0 = (sg * _SUPER + g) * _GROUP
            m = jnp.exp(x_ref[0:8, c0:c0 + _GROUP])
            s = m
            idx = jnp.zeros((8, _LANES), jnp.float32)
            for off in offs[:-1]:
                ek = jnp.exp(x_ref[off:off + 8, c0:c0 + _GROUP])
                cond = ek > m
                m = jnp.maximum(m, ek)
                idx = jnp.where(cond, jnp.float32(off), idx)
                s = s + ek
            off = offs[-1]
            ek = jnp.exp(x_ref[off:off + 8, c0:c0 + _GROUP])
            cond = ek > m
            m = jnp.maximum(m, ek)
            idx = jnp.where(cond, jnp.float32(off), idx)
            s = s + jnp.where(sub8 >= jnp.float32(offs[-2] + 8 - off),
                              ek, 0.0)
            cls = idx + sub8
            for sh in (4, 2, 1):
                mr = pltpu.roll(m, sh, 0)
                cr = pltpu.roll(cls, sh, 0)
                cond = mr > m
                m = jnp.maximum(m, mr)
                cls = jnp.where(cond, cr, cls)
                sr = pltpu.roll(s, sh, 0)
                s = s + sr
            mx = m[0:1, :]
            pidx = cls[0:1, :]
            ssum = s[0:1, :]
            labf = lab_ref[pl.ds(c0, _GROUP)].reshape(1, _GROUP).astype(jnp.float32)
            confrow_ref[pl.ds(g, 1), :] = mx / ssum
            accrow_ref[pl.ds(g, 1), :] = jnp.where(pidx == labf, 1.0, 0.0)
        conf8 = confrow_ref[...]
        acc8 = accrow_ref[...]
        base = i * _BLK + sg * _GROUP * _SUPER
        valid = (base + ridx) < n_total
        conf8 = jnp.where(valid, conf8, 0.0)
        for k in range(_N_BINS):
            gt = conf8 > bounds_ref[k]
            cnt_ref[k] += jnp.where(gt, 1.0, 0.0)
            sc_ref[k] += jnp.where(gt, conf8, 0.0)
            sa_ref[k] += jnp.where(gt, acc8, 0.0)

    @pl.when(i == n_steps - 1)
    def _fin():
        nf = jnp.float32(n_total)
        zero = jnp.zeros((1, 1), jnp.float32)
        gc = [jnp.sum(cnt_ref[k], keepdims=True)[:1, :1] for k in range(_N_BINS)]
        gs = [jnp.sum(sc_ref[k], keepdims=True)[:1, :1] for k in range(_N_BINS)]
        ga = [jnp.sum(sa_ref[k], keepdims=True)[:1, :1] for k in range(_N_BINS)]
        gc.append(zero)
        gs.append(zero)
        ga.append(zero)
        ece = zero
        oe = zero
        for b in range(_N_BINS):
            cnt = gc[b] - gc[b + 1]
            sc = gs[b] - gs[b + 1]
            sa = ga[b] - ga[b + 1]
            prop = cnt / nf
            denom = jnp.maximum(cnt, 1.0)
            accb = sa / denom
            avgc = sc / denom
            ce = avgc - accb
            ece = ece + jnp.abs(ce) * prop
            oe = oe + avgc * jnp.maximum(ce, 0.0) * prop
            accb_ref[pl.ds(b, 1), :] = accb
            prop_ref[pl.ds(b, 1), :] = prop
            ce_ref[pl.ds(b, 1), :] = jnp.abs(ce)
        ece_ref[...] = ece
        oe_ref[...] = oe


def kernel(logits, labels):
    n, c = logits.shape
    steps = pl.cdiv(n, _BLK)
    bounds = jnp.linspace(0.0, 1.0, _N_BINS + 1)
    lt = logits.T

    outs = pl.pallas_call(
        functools.partial(_ece_kernel, n_total=n, n_cols=c, n_steps=steps),
        grid=(steps,),
        in_specs=[
            pl.BlockSpec(memory_space=pltpu.SMEM),
            pl.BlockSpec((c, _BLK), lambda i: (0, i)),
            pl.BlockSpec((_BLK,), lambda i: (i,)),
        ],
        out_specs=[
            pl.BlockSpec((1, 1), lambda i: (0, 0)),
            pl.BlockSpec((_N_BINS, 1), lambda i: (0, 0)),
            pl.BlockSpec((1, 1), lambda i: (0, 0)),
            pl.BlockSpec((_N_BINS, 1), lambda i: (0, 0)),
            pl.BlockSpec((_N_BINS, 1), lambda i: (0, 0)),
        ],
        out_shape=[
            jax.ShapeDtypeStruct((1, 1), jnp.float32),
            jax.ShapeDtypeStruct((_N_BINS, 1), jnp.float32),
            jax.ShapeDtypeStruct((1, 1), jnp.float32),
            jax.ShapeDtypeStruct((_N_BINS, 1), jnp.float32),
            jax.ShapeDtypeStruct((_N_BINS, 1), jnp.float32),
        ],
        scratch_shapes=[pltpu.VMEM((_N_BINS, 8, _LANES), jnp.float32)] * 3
        + [pltpu.VMEM((8, _LANES), jnp.float32)] * 2,
        compiler_params=pltpu.CompilerParams(dimension_semantics=("arbitrary",)),
    )(bounds, lt, labels)
    ece, accb, oe, prop, ce = outs
    return (ece.reshape(()), accb[:, 0], oe.reshape(()), prop[:, 0], ce[:, 0])
